# baseline (speedup 1.0000x reference)
import functools

import jax
import jax.numpy as jnp
from jax import lax
from jax.experimental import pallas as pl
from jax.experimental.pallas import tpu as pltpu
from jax.experimental.pallas import tpu_sc as plsc

S = 2048
B = 4
D = 128
N = S * B
NW = 32
RPW = N // NW
SPW = S // NW
LANES = 16
NCHUNK = 4
RPC = RPW // NCHUNK
SPC = SPW // NCHUNK


def _emb_body(x_hbm, pe_hbm, table_hbm, out_hbm, idx_v, rows_v, pe_v, sems):
    wid = lax.axis_index("s") * 2 + lax.axis_index("c")
    base = wid * RPW
    s0 = wid * SPW

    pltpu.sync_copy(x_hbm.at[pl.ds(wid * 2, 2)], idx_v)
    gathers = []
    for c in range(NCHUNK):
        gathers.append(pltpu.async_copy(
            table_hbm.at[idx_v.at[c // 2, pl.ds((c % 2) * RPC, RPC)]],
            rows_v.at[pl.ds(c * RPC, RPC)],
            sems.at[c],
        ))
    pltpu.sync_copy(pe_hbm.at[pl.ds(s0, SPW)], pe_v)

    def add_body(r, _):
        row = r * B
        for j in range(D // LANES):
            sl = pl.ds(j * LANES, LANES)
            p = pe_v[r, 0, sl]
            for b in range(B):
                rows_v[row + b, sl] = rows_v[row + b, sl] + p
        return _

    outs = []
    for c in range(NCHUNK):
        gathers[c].wait()
        lax.fori_loop(c * SPC, (c + 1) * SPC, add_body, None)
        outs.append(pltpu.async_copy(
            rows_v.at[pl.ds(c * RPC, RPC)],
            out_hbm.at[pl.ds(base + c * RPC, RPC)],
            sems.at[NCHUNK + c],
        ))
    for o in outs:
        o.wait()


@jax.jit
def _emb(x2, pe, table):
    mesh = plsc.VectorSubcoreMesh(core_axis_name="c", subcore_axis_name="s")
    f = functools.partial(
        pl.kernel,
        mesh=mesh,
        out_type=jax.ShapeDtypeStruct((N, D), jnp.float32),
        scratch_types=[
            pltpu.VMEM((2, 128), jnp.int32),
            pltpu.VMEM((RPW, D), jnp.float32),
            pltpu.VMEM((SPW, 1, D), jnp.float32),
            pltpu.SemaphoreType.DMA((2 * NCHUNK,)),
        ],
    )(_emb_body)
    return f(x2, pe, table)


def kernel(x, table, pe):
    x2 = x.reshape(N // 128, 128)
    out = _emb(x2, pe, table)
    return out.reshape(S, B, D)


# speedup vs baseline: 1.0640x; 1.0640x over previous
import functools

import jax
import jax.numpy as jnp
from jax import lax
from jax.experimental import pallas as pl
from jax.experimental.pallas import tpu as pltpu
from jax.experimental.pallas import tpu_sc as plsc

S = 2048
B = 4
D = 128
N = S * B
NW = 32
RPW = N // NW
SPW = S // NW
LANES = 16
NCHUNK = 4
RPC = RPW // NCHUNK
SPC = SPW // NCHUNK


def _emb_body(x_hbm, pe_hbm, table_hbm, out_hbm, idx_v, rows_v, pe_v, sems):
    wid = lax.axis_index("s") * 2 + lax.axis_index("c")
    base = wid * RPW
    s0 = wid * SPW

    pltpu.sync_copy(x_hbm.at[pl.ds(base, RPW)], idx_v)
    gathers = []
    for c in range(NCHUNK):
        gathers.append(pltpu.async_copy(
            table_hbm.at[idx_v.at[pl.ds(c * RPC, RPC)]],
            rows_v.at[pl.ds(c * RPC, RPC)],
            sems.at[c],
        ))
    pltpu.sync_copy(pe_hbm.at[pl.ds(s0, SPW)], pe_v)

    outs = []
    for c in range(NCHUNK):
        gathers[c].wait()

        @plsc.parallel_loop(c * SPC, (c + 1) * SPC, unroll=4)
        def add_body(r):
            row = r * B
            for j in range(D // LANES):
                sl = pl.ds(j * LANES, LANES)
                p = pe_v[r, 0, sl]
                for b in range(B):
                    rows_v[row + b, sl] = rows_v[row + b, sl] + p

        outs.append(pltpu.async_copy(
            rows_v.at[pl.ds(c * RPC, RPC)],
            out_hbm.at[pl.ds(base + c * RPC, RPC)],
            sems.at[NCHUNK + c],
        ))
    for o in outs:
        o.wait()


@jax.jit
def _emb(x2, pe, table):
    mesh = plsc.VectorSubcoreMesh(core_axis_name="c", subcore_axis_name="s")
    f = functools.partial(
        pl.kernel,
        mesh=mesh,
        out_type=jax.ShapeDtypeStruct((N, D), jnp.float32),
        scratch_types=[
            pltpu.VMEM((RPW,), jnp.int32),
            pltpu.VMEM((RPW, D), jnp.float32),
            pltpu.VMEM((SPW, 1, D), jnp.float32),
            pltpu.SemaphoreType.DMA((2 * NCHUNK,)),
        ],
    )(_emb_body)
    return f(x2, pe, table)


def kernel(x, table, pe):
    x1 = x.reshape(N)
    out = _emb(x1, pe, table)
    return out.reshape(S, B, D)


# speedup vs baseline: 1.1105x; 1.0437x over previous
import functools

import jax
import jax.numpy as jnp
from jax import lax
from jax.experimental import pallas as pl
from jax.experimental.pallas import tpu as pltpu
from jax.experimental.pallas import tpu_sc as plsc

S = 2048
B = 4
D = 128
N = S * B
NW = 32
RPW = N // NW
SPW = S // NW
LANES = 16
NCHUNK = 2
RPC = RPW // NCHUNK
SPC = SPW // NCHUNK


def _emb_body(x_hbm, pe_hbm, table_hbm, out_hbm, idx_v, rows_v, pe_v, sems):
    wid = lax.axis_index("s") * 2 + lax.axis_index("c")
    base = wid * RPW
    s0 = wid * SPW

    pltpu.sync_copy(x_hbm.at[pl.ds(base, RPW)], idx_v)
    gathers = []
    for c in range(NCHUNK):
        gathers.append(pltpu.async_copy(
            table_hbm.at[idx_v.at[pl.ds(c * RPC, RPC)]],
            rows_v.at[pl.ds(c * RPC, RPC)],
            sems.at[c],
        ))
    pltpu.sync_copy(pe_hbm.at[pl.ds(s0, SPW)], pe_v)

    outs = []
    for c in range(NCHUNK):
        gathers[c].wait()

        @plsc.parallel_loop(c * SPC * 8, (c + 1) * SPC * 8, unroll=2)
        def add_body(t):
            r = t >> 3
            sl = pl.ds((t & 7) * LANES, LANES)
            row = r * B
            p = pe_v[r, 0, sl]
            for b in range(B):
                rows_v[row + b, sl] = rows_v[row + b, sl] + p

        outs.append(pltpu.async_copy(
            rows_v.at[pl.ds(c * RPC, RPC)],
            out_hbm.at[pl.ds(base + c * RPC, RPC)],
            sems.at[NCHUNK + c],
        ))
    for o in outs:
        o.wait()


@jax.jit
def _emb(x1, pe, table):
    mesh = plsc.VectorSubcoreMesh(core_axis_name="c", subcore_axis_name="s")
    f = functools.partial(
        pl.kernel,
        mesh=mesh,
        out_type=jax.ShapeDtypeStruct((N, D), jnp.float32),
        scratch_types=[
            pltpu.VMEM((RPW,), jnp.int32),
            pltpu.VMEM((RPW, D), jnp.float32),
            pltpu.VMEM((SPW, 1, D), jnp.float32),
            pltpu.SemaphoreType.DMA((2 * NCHUNK,)),
        ],
    )(_emb_body)
    return f(x1, pe, table)


def kernel(x, table, pe):
    x1 = x.reshape(N)
    out = _emb(x1, pe, table)
    return out.reshape(S, B, D)
